# baseline (speedup 1.0000x reference)
import functools

import jax
import jax.numpy as jnp
from jax import lax
from jax.experimental import pallas as pl
from jax.experimental.pallas import tpu as pltpu
from jax.experimental.pallas import tpu_sc as plsc

NC = 2
NS = 16
L = 16
NW = NC * NS
C = 32
NBUF = 3


def _emb_kernel(B, T, D, P, idx_hbm, wpe_hbm, wte_hbm, out_hbm,
                idx_v, wpe_v, rows_v, gsems, ssems, isems, psem):
    wid = lax.axis_index("s") * NC + lax.axis_index("c")
    pos_base = wid * P
    n_chunks = P // C
    n_jobs = n_chunks * B
    vecs_per_row = D // L

    idx_copies = [
        pltpu.async_copy(idx_hbm.at[pl.ds(b * T + pos_base, P)], idx_v.at[b],
                         isems.at[b])
        for b in range(B)
    ]
    wpe_copy = pltpu.async_copy(wpe_hbm.at[pl.ds(pos_base, P)], wpe_v, psem)

    gathers = [None] * n_jobs
    stores = [None] * n_jobs

    def fill_valu(j):
        pc = j // B
        buf = j % NBUF

        def fill_row(r, _):
            for v in range(vecs_per_row):
                sl = pl.ds(v * L, L)
                rows_v[buf, r, sl] = wpe_v[pc * C + r, sl]
            return _
        lax.fori_loop(0, C, fill_row, 0)

    def start_gather_add(j):
        pc, b = divmod(j, B)
        gathers[j] = pltpu.async_copy(
            wte_hbm.at[idx_v.at[b, pl.ds(pc * C, C)]], rows_v.at[j % NBUF],
            gsems.at[j % NBUF], add=True)

    wpe_copy.wait()
    for b in range(B):
        idx_copies[b].wait()
    for j in range(min(NBUF, n_jobs)):
        fill_valu(j)
        start_gather_add(j)
    for j in range(n_jobs):
        pc, b = divmod(j, B)
        gathers[j].wait()
        row_base = b * T + pos_base + pc * C
        stores[j] = pltpu.async_copy(
            rows_v.at[j % NBUF], out_hbm.at[pl.ds(row_base, C)],
            ssems.at[j % NBUF])
        jn = j + NBUF
        if jn < n_jobs:
            stores[j].wait()
            fill_valu(jn)
            start_gather_add(jn)
    for j in range(max(0, n_jobs - NBUF), n_jobs):
        stores[j].wait()


def kernel(idx, wte, wpe):
    B, T = idx.shape
    V, D = wte.shape
    P = T // NW

    mesh = plsc.VectorSubcoreMesh(core_axis_name="c", subcore_axis_name="s")
    body = functools.partial(_emb_kernel, B, T, D, P)
    out = pl.kernel(
        body,
        out_type=jax.ShapeDtypeStruct((B * T, D), jnp.float32),
        mesh=mesh,
        scratch_types=[
            pltpu.VMEM((B, P), jnp.int32),
            pltpu.VMEM((P, D), jnp.float32),
            pltpu.VMEM((NBUF, C, D), jnp.float32),
            pltpu.SemaphoreType.DMA((NBUF,)),
            pltpu.SemaphoreType.DMA((NBUF,)),
            pltpu.SemaphoreType.DMA((B,)),
            pltpu.SemaphoreType.DMA,
        ],
    )(idx.reshape(B * T), wte, wpe)
    return out.reshape(B, T, D)


# speedup vs baseline: 1.3265x; 1.3265x over previous
import functools

import jax
import jax.numpy as jnp
from jax import lax
from jax.experimental import pallas as pl
from jax.experimental.pallas import tpu as pltpu
from jax.experimental.pallas import tpu_sc as plsc

NC = 2
NS = 16
NW = NC * NS
C = 64
NBUF = 2


def _emb_kernel(B, T, D, P, idx_hbm, wpe_hbm, wte_hbm, out_hbm,
                idx_v, rows_v, wsems, gsems, ssems, isems):
    wid = lax.axis_index("s") * NC + lax.axis_index("c")
    pos_base = wid * P
    n_chunks = P // C
    n_jobs = n_chunks * B

    idx_copies = [
        pltpu.async_copy(idx_hbm.at[pl.ds(b * T + pos_base, P)], idx_v.at[b],
                         isems.at[b])
        for b in range(B)
    ]

    fills = [None] * n_jobs
    gathers = [None] * n_jobs
    stores = [None] * n_jobs

    def start_fill(j):
        pc = j // B
        fills[j] = pltpu.async_copy(
            wpe_hbm.at[pl.ds(pos_base + pc * C, C)], rows_v.at[j % NBUF],
            wsems.at[j % NBUF])

    def start_gather_add(j):
        pc, b = divmod(j, B)
        gathers[j] = pltpu.async_copy(
            wte_hbm.at[idx_v.at[b, pl.ds(pc * C, C)]], rows_v.at[j % NBUF],
            gsems.at[j % NBUF], add=True)

    for j in range(min(NBUF, n_jobs)):
        start_fill(j)
    for b in range(B):
        idx_copies[b].wait()
    for j in range(min(NBUF, n_jobs)):
        fills[j].wait()
        start_gather_add(j)
    for j in range(n_jobs):
        pc, b = divmod(j, B)
        gathers[j].wait()
        row_base = b * T + pos_base + pc * C
        stores[j] = pltpu.async_copy(
            rows_v.at[j % NBUF], out_hbm.at[pl.ds(row_base, C)],
            ssems.at[j % NBUF])
        jn = j + NBUF
        if jn < n_jobs:
            stores[j].wait()
            start_fill(jn)
            fills[jn].wait()
            start_gather_add(jn)
    for j in range(max(0, n_jobs - NBUF), n_jobs):
        stores[j].wait()


def kernel(idx, wte, wpe):
    B, T = idx.shape
    V, D = wte.shape
    P = T // NW

    mesh = plsc.VectorSubcoreMesh(core_axis_name="c", subcore_axis_name="s")
    body = functools.partial(_emb_kernel, B, T, D, P)
    out = pl.kernel(
        body,
        out_type=jax.ShapeDtypeStruct((B * T, D), jnp.float32),
        mesh=mesh,
        scratch_types=[
            pltpu.VMEM((B, P), jnp.int32),
            pltpu.VMEM((NBUF, C, D), jnp.float32),
            pltpu.SemaphoreType.DMA((NBUF,)),
            pltpu.SemaphoreType.DMA((NBUF,)),
            pltpu.SemaphoreType.DMA((NBUF,)),
            pltpu.SemaphoreType.DMA((B,)),
        ],
    )(idx.reshape(B * T), wte, wpe)
    return out.reshape(B, T, D)


# speedup vs baseline: 1.3365x; 1.0075x over previous
import functools

import jax
import jax.numpy as jnp
from jax import lax
from jax.experimental import pallas as pl
from jax.experimental.pallas import tpu as pltpu
from jax.experimental.pallas import tpu_sc as plsc

NC = 2
NS = 16
NW = NC * NS
C = 32
NBUF = 5


def _emb_kernel(B, T, D, P, idx_hbm, wpe_hbm, wte_hbm, out_hbm,
                idx_v, rows_v, wsems, gsems, ssems, isems):
    wid = lax.axis_index("s") * NC + lax.axis_index("c")
    pos_base = wid * P
    n_chunks = P // C
    n_jobs = n_chunks * B

    idx_copies = [
        pltpu.async_copy(idx_hbm.at[pl.ds(b * T + pos_base, P)], idx_v.at[b],
                         isems.at[b])
        for b in range(B)
    ]

    fills = [None] * n_jobs
    gathers = [None] * n_jobs
    stores = [None] * n_jobs

    def start_fill(j):
        pc = j // B
        fills[j] = pltpu.async_copy(
            wpe_hbm.at[pl.ds(pos_base + pc * C, C)], rows_v.at[j % NBUF],
            wsems.at[j % NBUF])

    def start_gather_add(j):
        pc, b = divmod(j, B)
        gathers[j] = pltpu.async_copy(
            wte_hbm.at[idx_v.at[b, pl.ds(pc * C, C)]], rows_v.at[j % NBUF],
            gsems.at[j % NBUF], add=True)

    for j in range(min(NBUF, n_jobs)):
        start_fill(j)
    for b in range(B):
        idx_copies[b].wait()
    for j in range(min(NBUF, n_jobs)):
        fills[j].wait()
        start_gather_add(j)
    for j in range(n_jobs):
        pc, b = divmod(j, B)
        gathers[j].wait()
        row_base = b * T + pos_base + pc * C
        stores[j] = pltpu.async_copy(
            rows_v.at[j % NBUF], out_hbm.at[pl.ds(row_base, C)],
            ssems.at[j % NBUF])
        jn = j + NBUF
        if jn < n_jobs:
            stores[j].wait()
            start_fill(jn)
            fills[jn].wait()
            start_gather_add(jn)
    for j in range(max(0, n_jobs - NBUF), n_jobs):
        stores[j].wait()


def kernel(idx, wte, wpe):
    B, T = idx.shape
    V, D = wte.shape
    P = T // NW

    mesh = plsc.VectorSubcoreMesh(core_axis_name="c", subcore_axis_name="s")
    body = functools.partial(_emb_kernel, B, T, D, P)
    out = pl.kernel(
        body,
        out_type=jax.ShapeDtypeStruct((B * T, D), jnp.float32),
        mesh=mesh,
        scratch_types=[
            pltpu.VMEM((B, P), jnp.int32),
            pltpu.VMEM((NBUF, C, D), jnp.float32),
            pltpu.SemaphoreType.DMA((NBUF,)),
            pltpu.SemaphoreType.DMA((NBUF,)),
            pltpu.SemaphoreType.DMA((NBUF,)),
            pltpu.SemaphoreType.DMA((B,)),
        ],
    )(idx.reshape(B * T), wte, wpe)
    return out.reshape(B, T, D)
